# baseline (speedup 1.0000x reference)
import functools

import jax
import jax.numpy as jnp
from jax import lax
from jax.experimental import pallas as pl
from jax.experimental.pallas import tpu as pltpu
from jax.experimental.pallas import tpu_sc as plsc

LANES = 16
CHUNK = 80


def kernel(indices, tables):
    T, B, G = indices.shape
    V, D = tables.shape[1], tables.shape[2]
    info = plsc.get_sparse_core_info()
    NC, NS = info.num_cores, info.num_subcores
    NW = NC * NS
    bpw = B // NW
    rpw = bpw * G
    nct = rpw // CHUNK
    nck = T * nct
    bpc = CHUNK // G
    dh = D // LANES
    grp = 128 // D

    gtab = tables.reshape((T * V) // grp, grp * D)
    idx_w = (
        indices.reshape(T, NW, rpw)
        .transpose(1, 0, 2)
        .reshape(NW, nck, CHUNK)
    )

    mesh = plsc.VectorSubcoreMesh(core_axis_name="c", subcore_axis_name="s")

    @functools.partial(
        pl.kernel,
        mesh=mesh,
        out_type=jax.ShapeDtypeStruct((B, T * D), jnp.float32),
        scratch_types=[
            pltpu.VMEM((nck, CHUNK), jnp.int32),
            pltpu.VMEM((2, CHUNK), jnp.int32),
            pltpu.VMEM((2, CHUNK), jnp.int32),
            pltpu.VMEM((2, CHUNK, grp * D), jnp.float32),
            pltpu.VMEM((bpw, T * D), jnp.float32),
            pltpu.SemaphoreType.DMA,
            pltpu.SemaphoreType.DMA,
        ],
    )
    def ebag(
        idx_hbm, tab_hbm, out_hbm, idx_v, gidx_v, off_v, gath_v, out_v, sem0, sem1
    ):
        wid = lax.axis_index("s") * NC + lax.axis_index("c")
        sems = (sem0, sem1)
        pltpu.sync_copy(idx_hbm.at[wid], idx_v)

        def issue(c, buf):
            t = c >> 3
            base = t * V
            for k in range(CHUNK // LANES):
                sl = pl.ds(k * LANES, LANES)
                x = idx_v[c, sl] + base
                gidx_v[buf, sl] = x >> 2
                off_v[buf, sl] = (x & (grp - 1)) * D
            pltpu.make_async_copy(
                tab_hbm.at[gidx_v.at[buf]], gath_v.at[buf], sems[buf]
            ).start()

        def drain(buf):
            pltpu.make_async_copy(
                tab_hbm.at[gidx_v.at[buf]], gath_v.at[buf], sems[buf]
            ).wait()

        def pool(c, buf):
            t = c >> 3
            s0 = (c & (nct - 1)) * bpc
            accs = {}
            for k in range(CHUNK // LANES):
                offs = off_v[buf, pl.ds(k * LANES, LANES)]
                for l in range(LANES):
                    r = k * LANES + l
                    b = r // G
                    off = offs[l]
                    for h in range(dh):
                        x = gath_v[buf, r, pl.ds(off + h * LANES, LANES)]
                        key = (b, h)
                        accs[key] = x if key not in accs else accs[key] + x
            for b in range(bpc):
                for h in range(dh):
                    out_v[s0 + b, pl.ds(t * D + h * LANES, LANES)] = accs[(b, h)]

        issue(0, 0)

        def pair_body(i, carry):
            c0 = 2 * i
            c1 = c0 + 1
            issue(c1, 1)
            drain(0)
            pool(c0, 0)

            @pl.when(c0 + 2 < nck)
            def _():
                issue(c0 + 2, 0)

            drain(1)
            pool(c1, 1)
            return carry

        lax.fori_loop(0, nck // 2, pair_body, 0)
        pltpu.sync_copy(out_v, out_hbm.at[pl.ds(wid * bpw, bpw)])

    return ebag(idx_w, gtab)
